# baseline (device time: 60917 ns/iter reference)
import jax
import jax.numpy as jnp
from jax import lax
from jax.experimental import pallas as pl
from jax.experimental.pallas import tpu as pltpu

N_DEV = 8
B, SQ, SKV, HQ_LOC, DH = 2, 128, 128, 4, 64
M = B * SQ
D_MODEL = 512
D_HEADS = HQ_LOC * DH


def _block_mask():
    qi = lax.broadcasted_iota(jnp.int32, (SQ, SKV), 0) // 64
    ki = lax.broadcasted_iota(jnp.int32, (SQ, SKV), 1) // 64
    return (qi == ki) | (ki == 0) | ((qi + ki) % 3 == 0)


def kernel(x, Wq, K_ext, V_ext, Wo):
    my = lax.axis_index("i")
    K_loc = lax.dynamic_slice_in_dim(K_ext, my * HQ_LOC, HQ_LOC, axis=2)
    V_loc = lax.dynamic_slice_in_dim(V_ext, my * HQ_LOC, HQ_LOC, axis=2)
    K_loc = K_loc.transpose(0, 2, 1, 3)
    V_loc = V_loc.transpose(0, 2, 1, 3)
    x2d = x.reshape(M, D_MODEL)

    def body(x_ref, wq_ref, k_ref, v_ref, wo_ref, out_ref,
             ctx_ref, comm_ref, send_sems, recv_sems):
        my_pos = lax.axis_index("i")
        left = (my_pos - 1) % N_DEV
        right = (my_pos + 1) % N_DEV

        barrier_sem = pltpu.get_barrier_semaphore()
        for nbr in [left, right]:
            pl.semaphore_signal(
                barrier_sem, inc=1,
                device_id=(nbr,), device_id_type=pl.DeviceIdType.MESH,
            )
        pl.semaphore_wait(barrier_sem, 2)

        q_all = jnp.dot(x_ref[...], wq_ref[...],
                        preferred_element_type=jnp.float32)
        mask = _block_mask()
        for b in range(B):
            for h in range(HQ_LOC):
                q = q_all[b * SQ:(b + 1) * SQ, h * DH:(h + 1) * DH]
                k = k_ref[b, h]
                v = v_ref[b, h]
                s = jnp.dot(q, k.T, preferred_element_type=jnp.float32) * 0.125
                s = jnp.where(mask, s, -1e9)
                m = jnp.max(s, axis=-1, keepdims=True)
                w = jnp.exp(s - m)
                w = w / jnp.sum(w, axis=-1, keepdims=True)
                ctx_ref[b * SQ:(b + 1) * SQ, h * DH:(h + 1) * DH] = jnp.dot(
                    w, v, preferred_element_type=jnp.float32)
        partial = jnp.dot(ctx_ref[...], wo_ref[...],
                          preferred_element_type=jnp.float32)
        out_ref[...] = partial
        comm_ref[0] = partial

        for hop in range(N_DEV - 1):
            send_slot = hop % 2
            recv_slot = (hop + 1) % 2
            rdma = pltpu.make_async_remote_copy(
                src_ref=comm_ref.at[send_slot],
                dst_ref=comm_ref.at[recv_slot],
                send_sem=send_sems.at[send_slot],
                recv_sem=recv_sems.at[recv_slot],
                device_id=(right,),
                device_id_type=pl.DeviceIdType.MESH,
            )
            rdma.start()
            rdma.wait()
            out_ref[...] += comm_ref[recv_slot]

    out2d = pl.pallas_call(
        body,
        out_shape=jax.ShapeDtypeStruct((M, D_MODEL), jnp.float32),
        in_specs=[pl.BlockSpec(memory_space=pltpu.VMEM)] * 5,
        out_specs=pl.BlockSpec(memory_space=pltpu.VMEM),
        scratch_shapes=[
            pltpu.VMEM((M, D_HEADS), jnp.float32),
            pltpu.VMEM((2, M, D_MODEL), jnp.float32),
            pltpu.SemaphoreType.DMA((2,)),
            pltpu.SemaphoreType.DMA((2,)),
        ],
        compiler_params=pltpu.CompilerParams(collective_id=0),
    )(x2d, Wq, K_loc, V_loc, Wo)
    return out2d.reshape(B, SQ, D_MODEL)


# device time: 30795 ns/iter; 1.9781x vs baseline; 1.9781x over previous
import jax
import jax.numpy as jnp
from jax import lax
from jax.experimental import pallas as pl
from jax.experimental.pallas import tpu as pltpu

N_DEV = 8
B, SQ, SKV, HQ_LOC, DH = 2, 128, 128, 4, 64
M = B * SQ
D_MODEL = 512
D_HEADS = HQ_LOC * DH


def _block_mask():
    qi = lax.broadcasted_iota(jnp.int32, (SQ, SKV), 0) // 64
    ki = lax.broadcasted_iota(jnp.int32, (SQ, SKV), 1) // 64
    return (qi == ki) | (ki == 0) | ((qi + ki) % 3 == 0)


def kernel(x, Wq, K_ext, V_ext, Wo):
    my = lax.axis_index("i")
    K_loc = lax.dynamic_slice_in_dim(K_ext, my * HQ_LOC, HQ_LOC, axis=2)
    V_loc = lax.dynamic_slice_in_dim(V_ext, my * HQ_LOC, HQ_LOC, axis=2)
    K_loc = K_loc.transpose(0, 2, 1, 3)
    V_loc = V_loc.transpose(0, 2, 1, 3)
    x2d = x.reshape(M, D_MODEL)

    MASKS = (1, 3, 4)

    def body(x_ref, wq_ref, k_ref, v_ref, wo_ref, out_ref,
             ctx_ref, recv_buf, send_sems, recv_sems):
        my_pos = lax.axis_index("i")
        partners = [my_pos ^ m for m in MASKS]

        barrier_sem = pltpu.get_barrier_semaphore()
        for nbr in partners:
            pl.semaphore_signal(
                barrier_sem, inc=1,
                device_id=(nbr,), device_id_type=pl.DeviceIdType.MESH,
            )
        pl.semaphore_wait(barrier_sem, len(MASKS))

        q_all = jnp.dot(x_ref[...], wq_ref[...],
                        preferred_element_type=jnp.float32)
        mask = _block_mask()
        for b in range(B):
            for h in range(HQ_LOC):
                q = q_all[b * SQ:(b + 1) * SQ, h * DH:(h + 1) * DH]
                k = k_ref[b, h]
                v = v_ref[b, h]
                s = jnp.dot(q, k.T, preferred_element_type=jnp.float32) * 0.125
                s = jnp.where(mask, s, -1e9)
                m = jnp.max(s, axis=-1, keepdims=True)
                w = jnp.exp(s - m)
                w = w / jnp.sum(w, axis=-1, keepdims=True)
                ctx_ref[b * SQ:(b + 1) * SQ, h * DH:(h + 1) * DH] = jnp.dot(
                    w, v, preferred_element_type=jnp.float32)
        partial = jnp.dot(ctx_ref[...], wo_ref[...],
                          preferred_element_type=jnp.float32)
        out_ref[...] = partial

        for r, partner in enumerate(partners):
            rdma = pltpu.make_async_remote_copy(
                src_ref=out_ref,
                dst_ref=recv_buf.at[r],
                send_sem=send_sems.at[r],
                recv_sem=recv_sems.at[r],
                device_id=(partner,),
                device_id_type=pl.DeviceIdType.MESH,
            )
            rdma.start()
            rdma.wait()
            out_ref[...] += recv_buf[r]

    out2d = pl.pallas_call(
        body,
        out_shape=jax.ShapeDtypeStruct((M, D_MODEL), jnp.float32),
        in_specs=[pl.BlockSpec(memory_space=pltpu.VMEM)] * 5,
        out_specs=pl.BlockSpec(memory_space=pltpu.VMEM),
        scratch_shapes=[
            pltpu.VMEM((M, D_HEADS), jnp.float32),
            pltpu.VMEM((3, M, D_MODEL), jnp.float32),
            pltpu.SemaphoreType.DMA((3,)),
            pltpu.SemaphoreType.DMA((3,)),
        ],
        compiler_params=pltpu.CompilerParams(collective_id=0),
    )(x2d, Wq, K_loc, V_loc, Wo)
    return out2d.reshape(B, SQ, D_MODEL)


# device time: 22441 ns/iter; 2.7145x vs baseline; 1.3723x over previous
import jax
import jax.numpy as jnp
from jax import lax
from jax.experimental import pallas as pl
from jax.experimental.pallas import tpu as pltpu

N_DEV = 8
B, SQ, SKV, HQ_LOC, DH = 2, 128, 128, 4, 64
M = B * SQ
D_MODEL = 512
D_HEADS = HQ_LOC * DH


def _block_mask():
    qi = lax.broadcasted_iota(jnp.int32, (SQ, SKV), 0) // 64
    ki = lax.broadcasted_iota(jnp.int32, (SQ, SKV), 1) // 64
    return (qi == ki) | (ki == 0) | ((qi + ki) % 3 == 0)


def kernel(x, Wq, K_ext, V_ext, Wo):
    my = lax.axis_index("i")
    K_loc = lax.dynamic_slice_in_dim(K_ext, my * HQ_LOC, HQ_LOC, axis=2)
    V_loc = lax.dynamic_slice_in_dim(V_ext, my * HQ_LOC, HQ_LOC, axis=2)
    K_loc = K_loc.transpose(0, 2, 1, 3)
    V_loc = V_loc.transpose(0, 2, 1, 3)
    x2d = x.reshape(M, D_MODEL)

    MASKS = (1, 3, 4)

    def body(x_ref, wq_ref, k_ref, v_ref, wo_ref, out_ref,
             ctx_ref, send_buf, recv_buf, send_sems, recv_sems):
        my_pos = lax.axis_index("i")
        partners = [my_pos ^ m for m in MASKS]

        barrier_sem = pltpu.get_barrier_semaphore()
        for nbr in partners:
            pl.semaphore_signal(
                barrier_sem, inc=1,
                device_id=(nbr,), device_id_type=pl.DeviceIdType.MESH,
            )
        pl.semaphore_wait(barrier_sem, len(MASKS))

        q_all = jnp.dot(x_ref[...], wq_ref[...],
                        preferred_element_type=jnp.float32)
        mask = _block_mask()
        for b in range(B):
            for h in range(HQ_LOC):
                q = q_all[b * SQ:(b + 1) * SQ, h * DH:(h + 1) * DH]
                k = k_ref[b, h]
                v = v_ref[b, h]
                s = jnp.dot(q, k.T, preferred_element_type=jnp.float32) * 0.125
                s = jnp.where(mask, s, -1e9)
                m = jnp.max(s, axis=-1, keepdims=True)
                w = jnp.exp(s - m)
                w = w / jnp.sum(w, axis=-1, keepdims=True)
                ctx_ref[b * SQ:(b + 1) * SQ, h * DH:(h + 1) * DH] = jnp.dot(
                    w, v, preferred_element_type=jnp.float32)
        partial = jnp.dot(ctx_ref[...], wo_ref[...],
                          preferred_element_type=jnp.float32)
        out_ref[...] = partial

        for r, partner in enumerate(partners):
            send_buf[r] = out_ref[...].astype(jnp.bfloat16)
            rdma = pltpu.make_async_remote_copy(
                src_ref=send_buf.at[r],
                dst_ref=recv_buf.at[r],
                send_sem=send_sems.at[r],
                recv_sem=recv_sems.at[r],
                device_id=(partner,),
                device_id_type=pl.DeviceIdType.MESH,
            )
            rdma.start()
            rdma.wait()
            out_ref[...] += recv_buf[r].astype(jnp.float32)

    out2d = pl.pallas_call(
        body,
        out_shape=jax.ShapeDtypeStruct((M, D_MODEL), jnp.float32),
        in_specs=[pl.BlockSpec(memory_space=pltpu.VMEM)] * 5,
        out_specs=pl.BlockSpec(memory_space=pltpu.VMEM),
        scratch_shapes=[
            pltpu.VMEM((M, D_HEADS), jnp.float32),
            pltpu.VMEM((3, M, D_MODEL), jnp.bfloat16),
            pltpu.VMEM((3, M, D_MODEL), jnp.bfloat16),
            pltpu.SemaphoreType.DMA((3,)),
            pltpu.SemaphoreType.DMA((3,)),
        ],
        compiler_params=pltpu.CompilerParams(collective_id=0),
    )(x2d, Wq, K_loc, V_loc, Wo)
    return out2d.reshape(B, SQ, D_MODEL)


# device time: 18309 ns/iter; 3.3272x vs baseline; 1.2257x over previous
import jax
import jax.numpy as jnp
from jax import lax
from jax.experimental import pallas as pl
from jax.experimental.pallas import tpu as pltpu

N_DEV = 8
B, SQ, SKV, HQ_LOC, DH = 2, 128, 128, 4, 64
M = B * SQ
D_MODEL = 512
D_HEADS = HQ_LOC * DH


def _block_mask():
    qi = lax.broadcasted_iota(jnp.int32, (SQ, SKV), 0) // 64
    ki = lax.broadcasted_iota(jnp.int32, (SQ, SKV), 1) // 64
    return (qi == ki) | (ki == 0) | ((qi + ki) % 3 == 0)


def kernel(x, Wq, K_ext, V_ext, Wo):
    my = lax.axis_index("i")
    K_loc = lax.dynamic_slice_in_dim(K_ext, my * HQ_LOC, HQ_LOC, axis=2)
    V_loc = lax.dynamic_slice_in_dim(V_ext, my * HQ_LOC, HQ_LOC, axis=2)
    K_loc = K_loc.transpose(0, 2, 1, 3)
    V_loc = V_loc.transpose(0, 2, 1, 3)
    x2d = x.reshape(M, D_MODEL)

    MASKS = (1, 3, 4)

    def body(x_ref, wq_ref, k_ref, v_ref, wo_ref, out_ref,
             ctx_ref, send_buf, recv_buf, send_sems, recv_sems):
        my_pos = lax.axis_index("i")
        partners = [my_pos ^ m for m in MASKS]

        barrier_sem = pltpu.get_barrier_semaphore()
        for nbr in partners:
            pl.semaphore_signal(
                barrier_sem, inc=1,
                device_id=(nbr,), device_id_type=pl.DeviceIdType.MESH,
            )
        pl.semaphore_wait(barrier_sem, len(MASKS))

        q_all = jnp.dot(x_ref[...], wq_ref[...],
                        preferred_element_type=jnp.float32)
        mask = _block_mask()
        for b in range(B):
            for h in range(HQ_LOC):
                q = q_all[b * SQ:(b + 1) * SQ, h * DH:(h + 1) * DH]
                k = k_ref[b, h]
                v = v_ref[b, h]
                s = jnp.dot(q, k.T, preferred_element_type=jnp.float32) * 0.125
                s = jnp.where(mask, s, -1e9)
                m = jnp.max(s, axis=-1, keepdims=True)
                w = jnp.exp(s - m)
                w = w / jnp.sum(w, axis=-1, keepdims=True)
                ctx_ref[b * SQ:(b + 1) * SQ, h * DH:(h + 1) * DH] = jnp.dot(
                    w, v, preferred_element_type=jnp.float32)
        HALF = D_MODEL // 2

        def start_rdma(r, m):
            partner = my_pos ^ MASKS[(r + m) % 3]
            rdma = pltpu.make_async_remote_copy(
                src_ref=send_buf.at[r, m],
                dst_ref=recv_buf.at[r, m],
                send_sem=send_sems.at[r, m],
                recv_sem=recv_sems.at[r, m],
                device_id=(partner,),
                device_id_type=pl.DeviceIdType.MESH,
            )
            rdma.start()
            return rdma

        rdmas = [[None, None] for _ in range(3)]
        for m in range(2):
            cols = slice(m * HALF, (m + 1) * HALF)
            pm = jnp.dot(ctx_ref[...], wo_ref[:, cols],
                         preferred_element_type=jnp.float32)
            out_ref[:, cols] = pm
            send_buf[0, m] = pm.astype(jnp.bfloat16)
            rdmas[0][m] = start_rdma(0, m)
        for r in range(3):
            for m in range(2):
                cols = slice(m * HALF, (m + 1) * HALF)
                rdmas[r][m].wait()
                acc = out_ref[:, cols] + recv_buf[r, m].astype(jnp.float32)
                out_ref[:, cols] = acc
                if r < 2:
                    send_buf[r + 1, m] = acc.astype(jnp.bfloat16)
                    rdmas[r + 1][m] = start_rdma(r + 1, m)

    out2d = pl.pallas_call(
        body,
        out_shape=jax.ShapeDtypeStruct((M, D_MODEL), jnp.float32),
        in_specs=[pl.BlockSpec(memory_space=pltpu.VMEM)] * 5,
        out_specs=pl.BlockSpec(memory_space=pltpu.VMEM),
        scratch_shapes=[
            pltpu.VMEM((M, D_HEADS), jnp.float32),
            pltpu.VMEM((3, 2, M, D_MODEL // 2), jnp.bfloat16),
            pltpu.VMEM((3, 2, M, D_MODEL // 2), jnp.bfloat16),
            pltpu.SemaphoreType.DMA((3, 2)),
            pltpu.SemaphoreType.DMA((3, 2)),
        ],
        compiler_params=pltpu.CompilerParams(collective_id=0),
    )(x2d, Wq, K_loc, V_loc, Wo)
    return out2d.reshape(B, SQ, D_MODEL)


# device time: 16918 ns/iter; 3.6007x vs baseline; 1.0822x over previous
import jax
import jax.numpy as jnp
from jax import lax
from jax.experimental import pallas as pl
from jax.experimental.pallas import tpu as pltpu

N_DEV = 8
B, SQ, SKV, HQ_LOC, DH = 2, 128, 128, 4, 64
M = B * SQ
D_MODEL = 512
D_HEADS = HQ_LOC * DH


def _block_mask():
    qi = lax.broadcasted_iota(jnp.int32, (SQ, SKV), 0) // 64
    ki = lax.broadcasted_iota(jnp.int32, (SQ, SKV), 1) // 64
    return (qi == ki) | (ki == 0) | ((qi + ki) % 3 == 0)


def kernel(x, Wq, K_ext, V_ext, Wo):
    my = lax.axis_index("i")
    K_loc = lax.dynamic_slice_in_dim(K_ext, my * HQ_LOC, HQ_LOC, axis=2)
    V_loc = lax.dynamic_slice_in_dim(V_ext, my * HQ_LOC, HQ_LOC, axis=2)
    K_loc = K_loc.transpose(0, 2, 1, 3)
    V_loc = V_loc.transpose(0, 2, 1, 3)
    x2d = x.reshape(M, D_MODEL)

    MASKS = (1, 3, 4)

    def body(x_ref, wq_ref, k_ref, v_ref, wo_ref, out_ref,
             ctx_ref, send_buf, recv_buf, send_sems, recv_sems):
        my_pos = lax.axis_index("i")
        partners = [my_pos ^ m for m in MASKS]

        barrier_sem = pltpu.get_barrier_semaphore()
        for nbr in partners:
            pl.semaphore_signal(
                barrier_sem, inc=1,
                device_id=(nbr,), device_id_type=pl.DeviceIdType.MESH,
            )
        pl.semaphore_wait(barrier_sem, len(MASKS))

        HALF = D_MODEL // 2
        mask = _block_mask()

        def start_rdma(r, m, b):
            partner = my_pos ^ MASKS[(r + m) % 3]
            rdma = pltpu.make_async_remote_copy(
                src_ref=send_buf.at[r, m, b],
                dst_ref=recv_buf.at[r, m, b],
                send_sem=send_sems.at[r, m, b],
                recv_sem=recv_sems.at[r, m, b],
                device_id=(partner,),
                device_id_type=pl.DeviceIdType.MESH,
            )
            rdma.start()
            return rdma

        rdmas = {}
        for b in range(B):
            rows = slice(b * SQ, (b + 1) * SQ)
            q_b = jnp.dot(x_ref[rows, :], wq_ref[...],
                          preferred_element_type=jnp.float32)
            for h in range(HQ_LOC):
                q = q_b[:, h * DH:(h + 1) * DH]
                k = k_ref[b, h]
                v = v_ref[b, h]
                s = jnp.dot(q, k.T, preferred_element_type=jnp.float32) * 0.125
                s = jnp.where(mask, s, -1e9)
                mx = jnp.max(s, axis=-1, keepdims=True)
                w = jnp.exp(s - mx)
                w = w / jnp.sum(w, axis=-1, keepdims=True)
                ctx_ref[rows, h * DH:(h + 1) * DH] = jnp.dot(
                    w, v, preferred_element_type=jnp.float32)
            for m in range(2):
                cols = slice(m * HALF, (m + 1) * HALF)
                pm = jnp.dot(ctx_ref[rows, :], wo_ref[:, cols],
                             preferred_element_type=jnp.float32)
                out_ref[rows, cols] = pm
                send_buf[0, m, b] = pm.astype(jnp.bfloat16)
                rdmas[(0, m, b)] = start_rdma(0, m, b)
        for r in range(3):
            for b in range(B):
                rows = slice(b * SQ, (b + 1) * SQ)
                for m in range(2):
                    cols = slice(m * HALF, (m + 1) * HALF)
                    rdmas[(r, m, b)].wait()
                    acc = out_ref[rows, cols] + recv_buf[r, m, b].astype(
                        jnp.float32)
                    out_ref[rows, cols] = acc
                    if r < 2:
                        send_buf[r + 1, m, b] = acc.astype(jnp.bfloat16)
                        rdmas[(r + 1, m, b)] = start_rdma(r + 1, m, b)

    out2d = pl.pallas_call(
        body,
        out_shape=jax.ShapeDtypeStruct((M, D_MODEL), jnp.float32),
        in_specs=[pl.BlockSpec(memory_space=pltpu.VMEM)] * 5,
        out_specs=pl.BlockSpec(memory_space=pltpu.VMEM),
        scratch_shapes=[
            pltpu.VMEM((M, D_HEADS), jnp.float32),
            pltpu.VMEM((3, 2, B, SQ, D_MODEL // 2), jnp.bfloat16),
            pltpu.VMEM((3, 2, B, SQ, D_MODEL // 2), jnp.bfloat16),
            pltpu.SemaphoreType.DMA((3, 2, B)),
            pltpu.SemaphoreType.DMA((3, 2, B)),
        ],
        compiler_params=pltpu.CompilerParams(collective_id=0),
    )(x2d, Wq, K_loc, V_loc, Wo)
    return out2d.reshape(B, SQ, D_MODEL)
